# baseline (speedup 1.0000x reference)
import functools

import jax
import jax.numpy as jnp
from jax import lax
from jax.experimental import pallas as pl
from jax.experimental.pallas import tpu as pltpu
from jax.experimental.pallas import tpu_sc as plsc

N = 10000
D = 128
E = 320000
L = 4
NG = 256
TASKS = 12

SC_CORES = 2
SC_TILES = 16
W = SC_CORES * SC_TILES
CH = 128
BLK = 32
C0 = 160
EPW = 10240
EPAD = W * EPW
NCH = EPW // CH
NPAD = 10112
RPT = NPAD // SC_TILES

RB = 2000
NBLK = N // RB


def _segsum_body(h_hbm, src2_hbm, dst2_hbm, z_hbm, out_hbm,
                 sidx, didx, rows, acc, sem0, sem1):
    c = lax.axis_index("c")
    s = lax.axis_index("s")
    r0 = pl.multiple_of(s * RPT, 8)

    @pl.when(c == 0)
    def _():
        pltpu.sync_copy(z_hbm.at[pl.ds(r0, RPT)], acc.at[pl.ds(r0, RPT)])
        plsc.subcore_barrier()

        sems = (sem0, sem1)

        for blk in range(C0 // BLK):
            row0 = pl.multiple_of(s * C0 + blk * BLK, 8)
            pltpu.sync_copy(src2_hbm.at[pl.ds(row0, BLK)], sidx)
            pltpu.sync_copy(dst2_hbm.at[pl.ds(row0, BLK)], didx)
            for b in range(2):
                pltpu.async_copy(h_hbm.at[sidx.at[b]], rows.at[b], sems[b])

            def body(j, carry):
                i0 = j * 2
                for b in range(2):
                    i = i0 + b
                    pltpu.make_async_copy(h_hbm.at[sidx.at[i]], rows.at[b],
                                          sems[b]).wait()
                    pltpu.sync_copy(rows.at[b], acc.at[didx.at[i]], add=True)

                    @pl.when(i + 2 < BLK)
                    def _():
                        pltpu.async_copy(h_hbm.at[sidx.at[i + 2]], rows.at[b],
                                         sems[b])

                return carry

            lax.fori_loop(0, BLK // 2, body, 0)

        plsc.subcore_barrier()
        pltpu.sync_copy(acc.at[pl.ds(r0, RPT)], out_hbm.at[pl.ds(r0, RPT)])


@functools.cache
def _make_segsum():
    return pl.kernel(
        _segsum_body,
        out_type=jax.ShapeDtypeStruct((NPAD, D), jnp.float32),
        mesh=plsc.VectorSubcoreMesh(core_axis_name="c", subcore_axis_name="s"),
        scratch_types=[
            pltpu.VMEM((BLK, CH), jnp.int32),
            pltpu.VMEM((BLK, CH), jnp.int32),
            pltpu.VMEM((2, CH, D), jnp.float32),
            pltpu.VMEM_SHARED((NPAD, D), jnp.float32),
            pltpu.SemaphoreType.DMA,
            pltpu.SemaphoreType.DMA,
        ],
    )


def _segsum(h, src_p, dst_p, zeros):
    return _make_segsum()(h, src_p.reshape(EPAD // CH, CH),
                          dst_p.reshape(EPAD // CH, CH), zeros)


def _mlp1_body(h_ref, p_ref, w_ref, b_ref, out_ref, st_ref):
    i = pl.program_id(0)
    t = h_ref[...] + p_ref[...]
    h1 = jnp.dot(t.astype(jnp.bfloat16), w_ref[...].astype(jnp.bfloat16),
                 preferred_element_type=jnp.float32) + b_ref[...]
    h1 = jnp.maximum(h1, 0.0)
    out_ref[...] = h1

    @pl.when(i == 0)
    def _():
        st_ref[...] = jnp.zeros_like(st_ref)

    st_ref[...] += jnp.sum(h1, axis=0, keepdims=True)


_mlp1 = pl.pallas_call(
    _mlp1_body,
    grid=(NBLK,),
    in_specs=[
        pl.BlockSpec((RB, D), lambda i: (i, 0)),
        pl.BlockSpec((RB, D), lambda i: (i, 0)),
        pl.BlockSpec((D, D), lambda i: (0, 0)),
        pl.BlockSpec((1, D), lambda i: (0, 0)),
    ],
    out_specs=[
        pl.BlockSpec((RB, D), lambda i: (i, 0)),
        pl.BlockSpec((1, D), lambda i: (0, 0)),
    ],
    out_shape=[
        jax.ShapeDtypeStruct((N, D), jnp.float32),
        jax.ShapeDtypeStruct((1, D), jnp.float32),
    ],
)


def _css_body(x_ref, s_ref, out_ref):
    i = pl.program_id(0)
    m = s_ref[...] / N

    @pl.when(i == 0)
    def _():
        out_ref[...] = jnp.zeros_like(out_ref)

    d = x_ref[...] - m
    out_ref[...] += jnp.sum(d * d, axis=0, keepdims=True)


_css = pl.pallas_call(
    _css_body,
    grid=(NBLK,),
    in_specs=[
        pl.BlockSpec((RB, D), lambda i: (i, 0)),
        pl.BlockSpec((1, D), lambda i: (0, 0)),
    ],
    out_specs=pl.BlockSpec((1, D), lambda i: (0, 0)),
    out_shape=jax.ShapeDtypeStruct((1, D), jnp.float32),
)


def _bn(x, s_ref, c_ref, g_ref, t_ref):
    m = s_ref[...] / N
    v = c_ref[...] / N
    return (x - m) / jnp.sqrt(v + 1e-5) * g_ref[...] + t_ref[...]


def _mlp2_body(h1_ref, st_ref, c_ref, g_ref, t_ref, w_ref, b_ref,
               out_ref, st2_ref):
    i = pl.program_id(0)
    bn = _bn(h1_ref[...], st_ref, c_ref, g_ref, t_ref)
    h2 = jnp.dot(bn.astype(jnp.bfloat16), w_ref[...].astype(jnp.bfloat16),
                 preferred_element_type=jnp.float32) + b_ref[...]
    out_ref[...] = h2

    @pl.when(i == 0)
    def _():
        st2_ref[...] = jnp.zeros_like(st2_ref)

    st2_ref[...] += jnp.sum(h2, axis=0, keepdims=True)


_mlp2 = pl.pallas_call(
    _mlp2_body,
    grid=(NBLK,),
    in_specs=[
        pl.BlockSpec((RB, D), lambda i: (i, 0)),
        pl.BlockSpec((1, D), lambda i: (0, 0)),
        pl.BlockSpec((1, D), lambda i: (0, 0)),
        pl.BlockSpec((1, D), lambda i: (0, 0)),
        pl.BlockSpec((1, D), lambda i: (0, 0)),
        pl.BlockSpec((D, D), lambda i: (0, 0)),
        pl.BlockSpec((1, D), lambda i: (0, 0)),
    ],
    out_specs=[
        pl.BlockSpec((RB, D), lambda i: (i, 0)),
        pl.BlockSpec((1, D), lambda i: (0, 0)),
    ],
    out_shape=[
        jax.ShapeDtypeStruct((N, D), jnp.float32),
        jax.ShapeDtypeStruct((1, D), jnp.float32),
    ],
)


def _affine_body(h2_ref, st_ref, c_ref, g_ref, t_ref, out_ref):
    out_ref[...] = jnp.maximum(_bn(h2_ref[...], st_ref, c_ref, g_ref, t_ref), 0.0)


_affine = pl.pallas_call(
    _affine_body,
    grid=(NBLK,),
    in_specs=[
        pl.BlockSpec((RB, D), lambda i: (i, 0)),
        pl.BlockSpec((1, D), lambda i: (0, 0)),
        pl.BlockSpec((1, D), lambda i: (0, 0)),
        pl.BlockSpec((1, D), lambda i: (0, 0)),
        pl.BlockSpec((1, D), lambda i: (0, 0)),
    ],
    out_specs=pl.BlockSpec((RB, D), lambda i: (i, 0)),
    out_shape=jax.ShapeDtypeStruct((N, D), jnp.float32),
)


def _head_body(h_ref, b3_ref, wh1_ref, bh1_ref, wh2_ref, bh2_ref,
               out_ref, pool_ref):
    i = pl.program_id(0)

    @pl.when(i == 0)
    def _():
        pool_ref[...] = jnp.zeros_like(pool_ref)

    lbl = b3_ref[0]
    oh = (lax.broadcasted_iota(jnp.int32, (NG, RB), 0) == lbl).astype(jnp.float32)
    pool_ref[...] += jnp.dot(oh, h_ref[...], preferred_element_type=jnp.float32, precision=lax.Precision.HIGHEST)

    @pl.when(i == NBLK - 1)
    def _():
        z = jnp.dot(pool_ref[...].astype(jnp.bfloat16),
                    wh1_ref[...].astype(jnp.bfloat16),
                    preferred_element_type=jnp.float32) + bh1_ref[...]
        z = jnp.maximum(z, 0.0)
        out_ref[...] = jnp.dot(z.astype(jnp.bfloat16),
                               wh2_ref[...].astype(jnp.bfloat16),
                               preferred_element_type=jnp.float32) + bh2_ref[...]


_head = pl.pallas_call(
    _head_body,
    grid=(NBLK,),
    in_specs=[
        pl.BlockSpec((RB, D), lambda i: (i, 0)),
        pl.BlockSpec((1, 1, RB), lambda i: (i, 0, 0)),
        pl.BlockSpec((D, D), lambda i: (0, 0)),
        pl.BlockSpec((1, D), lambda i: (0, 0)),
        pl.BlockSpec((D, TASKS), lambda i: (0, 0)),
        pl.BlockSpec((1, TASKS), lambda i: (0, 0)),
    ],
    out_specs=pl.BlockSpec((NG, TASKS), lambda i: (0, 0)),
    out_shape=jax.ShapeDtypeStruct((NG, TASKS), jnp.float32),
    scratch_shapes=[pltpu.VMEM((NG, D), jnp.float32)],
)


def kernel(x, edge_index, edge_attr, batch,
           W1, B1, G1, T1, W2, B2, G2, T2, Wh1, bh1, Wh2, bh2):
    src = edge_index[0].astype(jnp.int32)
    dst = edge_index[1].astype(jnp.int32)
    pad = EPAD - E
    src_p = jnp.concatenate([src, jnp.zeros((pad,), jnp.int32)])
    dst_p = jnp.concatenate([dst, jnp.full((pad,), N, jnp.int32)])
    zeros = jnp.zeros((NPAD, D), jnp.float32)
    batch3 = batch.astype(jnp.int32).reshape(NBLK, 1, RB)

    h = x
    for l in range(L):
        parts = _segsum(h, src_p, dst_p, zeros)
        h1, s1 = _mlp1(h, parts, W1[l], B1[l].reshape(1, D))
        c1 = _css(h1, s1)
        h2, s2 = _mlp2(h1, s1, c1, G1[l].reshape(1, D), T1[l].reshape(1, D),
                       W2[l], B2[l].reshape(1, D))
        c2 = _css(h2, s2)
        h = _affine(h2, s2, c2, G2[l].reshape(1, D), T2[l].reshape(1, D))
    logits = _head(h, batch3, Wh1, bh1.reshape(1, D),
                   Wh2, bh2.reshape(1, TASKS))
    return logits


# speedup vs baseline: 1.2165x; 1.2165x over previous
import functools

import jax
import jax.numpy as jnp
from jax import lax
from jax.experimental import pallas as pl
from jax.experimental.pallas import tpu as pltpu
from jax.experimental.pallas import tpu_sc as plsc

N = 10000
D = 128
E = 320000
L = 4
NG = 256
TASKS = 12

SC_CORES = 2
SC_TILES = 16
CH = 128
BLK = 32
C0 = 128
C1 = 32
EPAD = SC_TILES * (C0 + C1) * CH
NPAD = 10112
RPT = NPAD // SC_TILES

RB = 2000
NBLK = N // RB


def _segsum_body(h_hbm, src2_hbm, dst2_hbm, out_hbm,
                 sidx, didx, rows, acc, sem0, sem1):
    c = lax.axis_index("c")
    s = lax.axis_index("s")
    r0 = pl.multiple_of(s * RPT, 8)

    def zbody(j, carry):
        rows[0, j, :16] = jnp.zeros((16,), jnp.float32)
        rows[0, j, 16:32] = jnp.zeros((16,), jnp.float32)
        rows[0, j, 32:48] = jnp.zeros((16,), jnp.float32)
        rows[0, j, 48:64] = jnp.zeros((16,), jnp.float32)
        rows[0, j, 64:80] = jnp.zeros((16,), jnp.float32)
        rows[0, j, 80:96] = jnp.zeros((16,), jnp.float32)
        rows[0, j, 96:112] = jnp.zeros((16,), jnp.float32)
        rows[0, j, 112:128] = jnp.zeros((16,), jnp.float32)
        return carry

    lax.fori_loop(0, CH, zbody, 0)
    for k in range(RPT // CH):
        pltpu.sync_copy(rows.at[0], acc.at[pl.ds(r0 + k * CH, CH)])
    pltpu.sync_copy(rows.at[0, pl.ds(0, RPT - (RPT // CH) * CH)],
                    acc.at[pl.ds(r0 + (RPT // CH) * CH,
                                 RPT - (RPT // CH) * CH)])
    plsc.subcore_barrier()

    sems = (sem0, sem1)

    def process(first_row, nblocks):
        for blk in range(nblocks):
            row0 = pl.multiple_of(first_row + blk * BLK, 8)
            pltpu.sync_copy(src2_hbm.at[pl.ds(row0, BLK)], sidx)
            pltpu.sync_copy(dst2_hbm.at[pl.ds(row0, BLK)], didx)
            for b in range(2):
                pltpu.async_copy(h_hbm.at[sidx.at[b]], rows.at[b], sems[b])

            def body(j, carry):
                i0 = j * 2
                for b in range(2):
                    i = i0 + b
                    pltpu.make_async_copy(h_hbm.at[sidx.at[i]], rows.at[b],
                                          sems[b]).wait()
                    pltpu.sync_copy(rows.at[b], acc.at[didx.at[i]], add=True)

                    @pl.when(i + 2 < BLK)
                    def _():
                        pltpu.async_copy(h_hbm.at[sidx.at[i + 2]], rows.at[b],
                                         sems[b])

                return carry

            lax.fori_loop(0, BLK // 2, body, 0)

    @pl.when(c == 0)
    def _():
        process(s * C0, C0 // BLK)

    @pl.when(c == 1)
    def _():
        process(SC_TILES * C0 + s * C1, C1 // BLK)

    plsc.subcore_barrier()
    pltpu.sync_copy(acc.at[pl.ds(r0, RPT)], out_hbm.at[c, pl.ds(r0, RPT)])


@functools.cache
def _make_segsum():
    return pl.kernel(
        _segsum_body,
        out_type=jax.ShapeDtypeStruct((SC_CORES, NPAD, D), jnp.float32),
        mesh=plsc.VectorSubcoreMesh(core_axis_name="c", subcore_axis_name="s"),
        scratch_types=[
            pltpu.VMEM((BLK, CH), jnp.int32),
            pltpu.VMEM((BLK, CH), jnp.int32),
            pltpu.VMEM((2, CH, D), jnp.float32),
            pltpu.VMEM_SHARED((NPAD, D), jnp.float32),
            pltpu.SemaphoreType.DMA,
            pltpu.SemaphoreType.DMA,
        ],
    )


def _segsum(h, src_p, dst_p):
    return _make_segsum()(h, src_p.reshape(EPAD // CH, CH),
                          dst_p.reshape(EPAD // CH, CH))


def _layer_body(h_ref, p_ref, w1_ref, b1_ref, g1_ref, t1_ref,
                w2_ref, b2_ref, g2_ref, t2_ref, out_ref,
                h1buf, h2buf, s1, c1, s2, c2):
    p = pl.program_id(0)
    i = pl.program_id(1)
    r = pl.ds(i * RB, RB)

    @pl.when(p == 0)
    def _():
        t = h_ref[...] + p_ref[0] + p_ref[1]
        h1 = jnp.dot(t.astype(jnp.bfloat16), w1_ref[...].astype(jnp.bfloat16),
                     preferred_element_type=jnp.float32) + b1_ref[...]
        h1 = jnp.maximum(h1, 0.0)
        h1buf[r, :] = h1

        @pl.when(i == 0)
        def _():
            s1[...] = jnp.zeros_like(s1)

        s1[...] += jnp.sum(h1, axis=0, keepdims=True)

    @pl.when(p == 1)
    def _():
        @pl.when(i == 0)
        def _():
            c1[...] = jnp.zeros_like(c1)

        d = h1buf[r, :] - s1[...] / N
        c1[...] += jnp.sum(d * d, axis=0, keepdims=True)

    @pl.when(p == 2)
    def _():
        bn = ((h1buf[r, :] - s1[...] / N) / jnp.sqrt(c1[...] / N + 1e-5)
              * g1_ref[...] + t1_ref[...])
        h2 = jnp.dot(bn.astype(jnp.bfloat16), w2_ref[...].astype(jnp.bfloat16),
                     preferred_element_type=jnp.float32) + b2_ref[...]
        h2buf[r, :] = h2

        @pl.when(i == 0)
        def _():
            s2[...] = jnp.zeros_like(s2)

        s2[...] += jnp.sum(h2, axis=0, keepdims=True)

    @pl.when(p == 3)
    def _():
        @pl.when(i == 0)
        def _():
            c2[...] = jnp.zeros_like(c2)

        d = h2buf[r, :] - s2[...] / N
        c2[...] += jnp.sum(d * d, axis=0, keepdims=True)

    @pl.when(p == 4)
    def _():
        bn = ((h2buf[r, :] - s2[...] / N) / jnp.sqrt(c2[...] / N + 1e-5)
              * g2_ref[...] + t2_ref[...])
        out_ref[...] = jnp.maximum(bn, 0.0)


def _in0(p, i):
    return (jnp.where(p == 0, i, NBLK - 1), 0)


def _in0p(p, i):
    return (0, jnp.where(p == 0, i, NBLK - 1), 0)


_const2 = lambda p, i: (0, 0)

_layer = pl.pallas_call(
    _layer_body,
    grid=(5, NBLK),
    in_specs=[
        pl.BlockSpec((RB, D), _in0),
        pl.BlockSpec((SC_CORES, RB, D), _in0p),
        pl.BlockSpec((D, D), _const2),
        pl.BlockSpec((1, D), _const2),
        pl.BlockSpec((1, D), _const2),
        pl.BlockSpec((1, D), _const2),
        pl.BlockSpec((D, D), _const2),
        pl.BlockSpec((1, D), _const2),
        pl.BlockSpec((1, D), _const2),
        pl.BlockSpec((1, D), _const2),
    ],
    out_specs=pl.BlockSpec((RB, D), lambda p, i: (jnp.where(p == 4, i, 0), 0)),
    out_shape=jax.ShapeDtypeStruct((N, D), jnp.float32),
    scratch_shapes=[
        pltpu.VMEM((N, D), jnp.float32),
        pltpu.VMEM((N, D), jnp.float32),
        pltpu.VMEM((1, D), jnp.float32),
        pltpu.VMEM((1, D), jnp.float32),
        pltpu.VMEM((1, D), jnp.float32),
        pltpu.VMEM((1, D), jnp.float32),
    ],
)


def _head_body(h_ref, b3_ref, wh1_ref, bh1_ref, wh2_ref, bh2_ref,
               out_ref, pool_ref):
    i = pl.program_id(0)

    @pl.when(i == 0)
    def _():
        pool_ref[...] = jnp.zeros_like(pool_ref)

    lbl = b3_ref[0]
    oh = (lax.broadcasted_iota(jnp.int32, (NG, RB), 0) == lbl).astype(jnp.float32)
    pool_ref[...] += jnp.dot(oh, h_ref[...], preferred_element_type=jnp.float32,
                             precision=lax.Precision.HIGHEST)

    @pl.when(i == NBLK - 1)
    def _():
        z = jnp.dot(pool_ref[...].astype(jnp.bfloat16),
                    wh1_ref[...].astype(jnp.bfloat16),
                    preferred_element_type=jnp.float32) + bh1_ref[...]
        z = jnp.maximum(z, 0.0)
        out_ref[...] = jnp.dot(z.astype(jnp.bfloat16),
                               wh2_ref[...].astype(jnp.bfloat16),
                               preferred_element_type=jnp.float32) + bh2_ref[...]


_head = pl.pallas_call(
    _head_body,
    grid=(NBLK,),
    in_specs=[
        pl.BlockSpec((RB, D), lambda i: (i, 0)),
        pl.BlockSpec((1, 1, RB), lambda i: (i, 0, 0)),
        pl.BlockSpec((D, D), lambda i: (0, 0)),
        pl.BlockSpec((1, D), lambda i: (0, 0)),
        pl.BlockSpec((D, TASKS), lambda i: (0, 0)),
        pl.BlockSpec((1, TASKS), lambda i: (0, 0)),
    ],
    out_specs=pl.BlockSpec((NG, TASKS), lambda i: (0, 0)),
    out_shape=jax.ShapeDtypeStruct((NG, TASKS), jnp.float32),
    scratch_shapes=[pltpu.VMEM((NG, D), jnp.float32)],
)


def kernel(x, edge_index, edge_attr, batch,
           W1, B1, G1, T1, W2, B2, G2, T2, Wh1, bh1, Wh2, bh2):
    src = edge_index[0].astype(jnp.int32)
    dst = edge_index[1].astype(jnp.int32)
    pad = EPAD - E
    src_p = jnp.concatenate([src, jnp.zeros((pad,), jnp.int32)])
    dst_p = jnp.concatenate([dst, jnp.full((pad,), N, jnp.int32)])
    batch3 = batch.astype(jnp.int32).reshape(NBLK, 1, RB)

    h = x
    for l in range(L):
        parts = _segsum(h, src_p, dst_p)
        h = _layer(h, parts, W1[l], B1[l].reshape(1, D),
                   G1[l].reshape(1, D), T1[l].reshape(1, D),
                   W2[l], B2[l].reshape(1, D),
                   G2[l].reshape(1, D), T2[l].reshape(1, D))
    logits = _head(h, batch3, Wh1, bh1.reshape(1, D),
                   Wh2, bh2.reshape(1, TASKS))
    return logits
